# Initial kernel scaffold; baseline (speedup 1.0000x reference)
import functools

import jax
import jax.numpy as jnp
from jax import lax
from jax.experimental import pallas as pl
from jax.experimental.pallas import tpu as pltpu
from jax.experimental.pallas import tpu_sc as plsc

VOCAB = 1_000_000
D = 64
L_SEQ = 200
BATCH = 4096
ROWS = BATCH * L_SEQ

NC, NS = 2, 16
NW = NC * NS
PER_W = ROWS // NW
CHUNK = 512
N_CHUNK = PER_W // CHUNK
GSZ = 128
N_G = CHUNK // GSZ


def _combo_tc_kernel(pos_ref, sent_ref, out_ref):
    p = pos_ref[...]
    s = sent_ref[...]
    out_ref[...] = (p[:, None, :] + s[None, :, :]).reshape(2 * L_SEQ, D)


def _build_combo(pos, sent):
    return pl.pallas_call(
        _combo_tc_kernel,
        out_shape=jax.ShapeDtypeStruct((2 * L_SEQ, D), jnp.float32),
    )(pos, sent)


def _sc_body(tok_hbm, seg_hbm, table_hbm, combo_hbm, out_hbm,
             tok_idx_v, seg_v, cmb_idx_v, rows_v, cmb_v, sem_t, sem_c):
    wid = lax.axis_index("s") * NC + lax.axis_index("c")
    base = wid * PER_W

    def chunk_body(ci, carry):
        off = base + ci * CHUNK
        blk = off // GSZ
        pltpu.sync_copy(tok_hbm.at[pl.ds(blk, N_G)], tok_idx_v)
        pltpu.sync_copy(seg_hbm.at[pl.ds(blk, N_G)], seg_v)
        for r in range(N_G):
            for c in range(GSZ // 16):
                g0 = off + r * GSZ + c * 16
                gpos = (g0 + lax.iota(jnp.int32, 16)) % L_SEQ
                cmb_idx_v[r, pl.ds(c * 16, 16)] = (
                    2 * gpos + seg_v[r, pl.ds(c * 16, 16)])
        cps = []
        for k in range(N_G):
            cps.append(pltpu.async_copy(
                table_hbm.at[tok_idx_v.at[k]],
                rows_v.at[pl.ds(k * GSZ, GSZ)], sem_t))
            cps.append(pltpu.async_copy(
                combo_hbm.at[cmb_idx_v.at[k]],
                cmb_v.at[pl.ds(k * GSZ, GSZ)], sem_c))
        for cp in cps:
            cp.wait()

        def add_body(r, carry2):
            for c in range(D // 16):
                sl = pl.ds(c * 16, 16)
                rows_v[r, sl] = rows_v[r, sl] + cmb_v[r, sl]
            return carry2
        lax.fori_loop(0, CHUNK, add_body, 0, unroll=2)

        pltpu.sync_copy(rows_v, out_hbm.at[pl.ds(off, CHUNK)])
        return carry

    lax.fori_loop(0, N_CHUNK, chunk_body, 0)


_sc_embed = functools.partial(
    pl.kernel,
    out_type=jax.ShapeDtypeStruct((ROWS, D), jnp.float32),
    mesh=plsc.VectorSubcoreMesh(core_axis_name="c", subcore_axis_name="s"),
    scratch_types=[
        pltpu.VMEM((N_G, GSZ), jnp.int32),
        pltpu.VMEM((N_G, GSZ), jnp.int32),
        pltpu.VMEM((N_G, GSZ), jnp.int32),
        pltpu.VMEM((CHUNK, D), jnp.float32),
        pltpu.VMEM((CHUNK, D), jnp.float32),
        pltpu.SemaphoreType.DMA,
        pltpu.SemaphoreType.DMA,
    ],
)(_sc_body)


def kernel(tokens, segment, token_embd_mat, position_embd_mat,
           sentence_embd_mat):
    tok = tokens.astype(jnp.int32).reshape(ROWS // GSZ, GSZ)
    seg = segment.astype(jnp.int32).reshape(ROWS // GSZ, GSZ)
    combo = _build_combo(position_embd_mat, sentence_embd_mat)
    out = _sc_embed(tok, seg, token_embd_mat, combo)
    return out.reshape(BATCH, L_SEQ, D)


# speedup vs baseline: 1.9183x; 1.9183x over previous
import functools

import jax
import jax.numpy as jnp
from jax import lax
from jax.experimental import pallas as pl
from jax.experimental.pallas import tpu as pltpu
from jax.experimental.pallas import tpu_sc as plsc

VOCAB = 1_000_000
D = 64
L_SEQ = 200
BATCH = 4096
ROWS = BATCH * L_SEQ

NC, NS = 2, 16
NW = NC * NS
PER_W = ROWS // NW
GSZ = 128
BLK = 1024
N_BLK = PER_W // BLK
CHUNK = 512
N_G = CHUNK // GSZ


def _combo_tc_kernel(pos_ref, sent_ref, out_ref):
    p = pos_ref[...]
    s = sent_ref[...]
    out_ref[...] = (p[:, None, :] + s[None, :, :]).reshape(2 * L_SEQ, D)


def _build_combo(pos, sent):
    return pl.pallas_call(
        _combo_tc_kernel,
        out_shape=jax.ShapeDtypeStruct((2 * L_SEQ, D), jnp.float32),
    )(pos, sent)


def _sc_body(tok_hbm, seg_hbm, table_hbm, combo_hbm, out_hbm,
             tok_idx_v, seg_v, cmb_idx_v, rows_v, cmb_v, sem_t, sem_c):
    wid = lax.axis_index("s") * NC + lax.axis_index("c")
    base = wid * PER_W

    def blk_body(bi, carry):
        boff = base + bi * BLK
        pltpu.sync_copy(tok_hbm.at[boff // BLK], tok_idx_v)
        pltpu.sync_copy(seg_hbm.at[boff // BLK], seg_v)
        for r in range(BLK // GSZ):
            for c in range(GSZ // 16):
                g0 = boff + r * GSZ + c * 16
                gpos = (g0 + lax.iota(jnp.int32, 16)) % L_SEQ
                cmb_idx_v[r, pl.ds(c * 16, 16)] = (
                    2 * gpos + seg_v[r, pl.ds(c * 16, 16)])
        for h in range(BLK // CHUNK):
            off = boff + h * CHUNK
            cps = []
            for k in range(N_G):
                kk = h * N_G + k
                cps.append(pltpu.async_copy(
                    table_hbm.at[tok_idx_v.at[kk]],
                    rows_v.at[pl.ds(k * GSZ, GSZ)], sem_t))
                cps.append(pltpu.async_copy(
                    combo_hbm.at[cmb_idx_v.at[kk]],
                    cmb_v.at[pl.ds(k * GSZ, GSZ)], sem_c))
            for cp in cps:
                cp.wait()

            def add_body(r, carry2):
                for c in range(D // 16):
                    sl = pl.ds(c * 16, 16)
                    rows_v[r, sl] = rows_v[r, sl] + cmb_v[r, sl]
                return carry2
            lax.fori_loop(0, CHUNK, add_body, 0, unroll=2)

            pltpu.sync_copy(rows_v, out_hbm.at[pl.ds(off, CHUNK)])
        return carry

    lax.fori_loop(0, N_BLK, blk_body, 0)


_sc_embed = functools.partial(
    pl.kernel,
    out_type=jax.ShapeDtypeStruct((ROWS, D), jnp.float32),
    mesh=plsc.VectorSubcoreMesh(core_axis_name="c", subcore_axis_name="s"),
    compiler_params=pltpu.CompilerParams(use_tc_tiling_on_sc=False),
    scratch_types=[
        pltpu.VMEM((BLK // GSZ, GSZ), jnp.int32),
        pltpu.VMEM((BLK // GSZ, GSZ), jnp.int32),
        pltpu.VMEM((BLK // GSZ, GSZ), jnp.int32),
        pltpu.VMEM((CHUNK, D), jnp.float32),
        pltpu.VMEM((CHUNK, D), jnp.float32),
        pltpu.SemaphoreType.DMA,
        pltpu.SemaphoreType.DMA,
    ],
)(_sc_body)


def kernel(tokens, segment, token_embd_mat, position_embd_mat,
           sentence_embd_mat):
    tok = tokens.astype(jnp.int32).reshape(ROWS // BLK, BLK // GSZ, GSZ)
    seg = segment.astype(jnp.int32).reshape(ROWS // BLK, BLK // GSZ, GSZ)
    combo = _build_combo(position_embd_mat, sentence_embd_mat)
    out = _sc_embed(tok, seg, token_embd_mat, combo)
    return out.reshape(BATCH, L_SEQ, D)


# speedup vs baseline: 2.1482x; 1.1199x over previous
import functools

import jax
import jax.numpy as jnp
from jax import lax
from jax.experimental import pallas as pl
from jax.experimental.pallas import tpu as pltpu
from jax.experimental.pallas import tpu_sc as plsc

VOCAB = 1_000_000
D = 64
L_SEQ = 200
BATCH = 4096
ROWS = BATCH * L_SEQ

NC, NS = 2, 16
NW = NC * NS
PER_W = ROWS // NW
GSZ = 128
BLK = 1024
N_BLK = PER_W // BLK
CHUNK = 256
N_CHUNKS = PER_W // CHUNK
N_G = CHUNK // GSZ


def _combo_tc_kernel(pos_ref, sent_ref, out_ref):
    p = pos_ref[...]
    s = sent_ref[...]
    out_ref[...] = (p[:, None, :] + s[None, :, :]).reshape(2 * L_SEQ, D)


def _build_combo(pos, sent):
    return pl.pallas_call(
        _combo_tc_kernel,
        out_shape=jax.ShapeDtypeStruct((2 * L_SEQ, D), jnp.float32),
    )(pos, sent)


def _sc_body(tok_hbm, seg_hbm, table_hbm, combo_hbm, out_hbm,
             tok_idx_v, cmb_idx_v, rows_v, cmb_v, sem_g, sem_c, sem_w):
    wid = lax.axis_index("s") * NC + lax.axis_index("c")
    base = wid * PER_W
    gb0 = wid * N_BLK
    chunks_per_blk = BLK // CHUNK

    def stage_block(lbi):
        slot = lbi % 2
        gb = gb0 + lbi
        pltpu.sync_copy(tok_hbm.at[gb], tok_idx_v.at[slot])
        pltpu.sync_copy(seg_hbm.at[gb], cmb_idx_v.at[slot])
        boff = base + lbi * BLK
        for r in range(BLK // GSZ):
            for c in range(GSZ // 16):
                sl = pl.ds(c * 16, 16)
                g0 = boff + r * GSZ + c * 16
                gpos = (g0 + lax.iota(jnp.int32, 16)) % L_SEQ
                cmb_idx_v[slot, r, sl] = 2 * gpos + cmb_idx_v[slot, r, sl]

    def gather_descs(ci):
        p = ci % 2
        slot = (ci // chunks_per_blk) % 2
        h = ci % chunks_per_blk
        ds = []
        for k in range(N_G):
            dst = pl.ds(k * GSZ, GSZ)
            ds.append(pltpu.make_async_copy(
                table_hbm.at[tok_idx_v.at[slot, h * N_G + k]],
                rows_v.at[p, dst], sem_g.at[p]))
            ds.append(pltpu.make_async_copy(
                combo_hbm.at[cmb_idx_v.at[slot, h * N_G + k]],
                cmb_v.at[p, dst], sem_c.at[p]))
        return ds

    def write_desc(ci):
        p = ci % 2
        return pltpu.make_async_copy(
            rows_v.at[p], out_hbm.at[pl.ds(base + ci * CHUNK, CHUNK)],
            sem_w.at[p])

    def fire_chunk(ci):
        for d in gather_descs(ci):
            d.start()

    def process_chunk(ci):
        p = ci % 2
        for d in gather_descs(ci):
            d.wait()

        def add_body(r, carry):
            for c in range(D // 16):
                sl = pl.ds(c * 16, 16)
                rows_v[p, r, sl] = rows_v[p, r, sl] + cmb_v[p, r, sl]
            return carry
        lax.fori_loop(0, CHUNK, add_body, 0, unroll=4)
        write_desc(ci).start()

    stage_block(0)
    fire_chunk(0)

    def loop_body(ci, carry):
        nci = ci + 1
        pl.when(nci % chunks_per_blk == 0)(
            lambda: stage_block(nci // chunks_per_blk))
        pl.when(nci >= 2)(lambda: write_desc(nci - 2).wait())
        fire_chunk(nci)
        process_chunk(ci)
        return carry

    lax.fori_loop(0, N_CHUNKS - 1, loop_body, 0)
    process_chunk(N_CHUNKS - 1)
    write_desc(N_CHUNKS - 2).wait()
    write_desc(N_CHUNKS - 1).wait()


_sc_embed = functools.partial(
    pl.kernel,
    out_type=jax.ShapeDtypeStruct((ROWS, D), jnp.float32),
    mesh=plsc.VectorSubcoreMesh(core_axis_name="c", subcore_axis_name="s"),
    compiler_params=pltpu.CompilerParams(use_tc_tiling_on_sc=False),
    scratch_types=[
        pltpu.VMEM((2, BLK // GSZ, GSZ), jnp.int32),
        pltpu.VMEM((2, BLK // GSZ, GSZ), jnp.int32),
        pltpu.VMEM((2, CHUNK, D), jnp.float32),
        pltpu.VMEM((2, CHUNK, D), jnp.float32),
        pltpu.SemaphoreType.DMA((2,)),
        pltpu.SemaphoreType.DMA((2,)),
        pltpu.SemaphoreType.DMA((2,)),
    ],
)(_sc_body)


def kernel(tokens, segment, token_embd_mat, position_embd_mat,
           sentence_embd_mat):
    tok = tokens.astype(jnp.int32).reshape(ROWS // BLK, BLK // GSZ, GSZ)
    seg = segment.astype(jnp.int32).reshape(ROWS // BLK, BLK // GSZ, GSZ)
    combo = _build_combo(position_embd_mat, sentence_embd_mat)
    out = _sc_embed(tok, seg, token_embd_mat, combo)
    return out.reshape(BATCH, L_SEQ, D)
